# Initial kernel scaffold; baseline (speedup 1.0000x reference)
import jax
import jax.numpy as jnp
from jax.experimental import pallas as pl


def kernel(x, edge_index, edge_attr, W, b):
    raise NotImplementedError("write your pallas kernel here")


# speedup vs baseline: 13.8049x; 13.8049x over previous
import dataclasses
import functools

import jax
import jax.numpy as jnp
from jax import lax
from jax.experimental import pallas as pl
from jax.experimental.pallas import tpu as pltpu
from jax.experimental.pallas import tpu_sc as plsc

N = 10000
D = 128
SEQ = 8
NC = 2
NS = 16
NW = NC * NS
K = 128
NP = 10240
SL = NP // NS

_mesh = plsc.VectorSubcoreMesh(core_axis_name="c", subcore_axis_name="s")

_sc_params = pltpu.CompilerParams()
if "needs_layout_passes" in pltpu.CompilerParams.__dataclass_fields__:
    _sc_params = dataclasses.replace(_sc_params, needs_layout_passes=False)


def _deg_body(dst_hbm, w_hbm, zn_hbm, deg_hbm, dstb, wb, degs):
    cid = lax.axis_index("c")
    sid = lax.axis_index("s")
    wid = cid * NS + sid
    nchunk = dst_hbm.shape[1]
    pltpu.sync_copy(dst_hbm.at[wid], dstb)
    pltpu.sync_copy(w_hbm.at[wid], wb)
    pltpu.sync_copy(zn_hbm.at[pl.ds(sid * SL, SL)], degs.at[pl.ds(sid * SL, SL)])
    plsc.subcore_barrier()

    @pl.loop(0, nchunk)
    def _(c):
        pltpu.sync_copy(wb.at[c], degs.at[dstb.at[c]], add=True)

    plsc.subcore_barrier()
    pltpu.sync_copy(degs.at[pl.ds(sid * SL, SL)],
                    deg_hbm.at[cid, pl.ds(sid * SL, SL)])


def _sc_degree(dst3, w3, nchunk):
    kern = functools.partial(
        pl.kernel,
        out_type=jax.ShapeDtypeStruct((NC, NP), jnp.float32),
        mesh=_mesh,
        scratch_types=[
            pltpu.VMEM((nchunk, K), jnp.int32),
            pltpu.VMEM((nchunk, K), jnp.float32),
            pltpu.VMEM_SHARED((NP,), jnp.float32),
        ],
    )(_deg_body)
    zn = jnp.zeros((NP,), jnp.float32)
    return kern(dst3, w3, zn)


def _agg_body(src_hbm, dst_hbm, w_hbm, g_hbm, znd_hbm, out_hbm,
              srcb, dstb, wb, rows, accs):
    cid = lax.axis_index("c")
    sid = lax.axis_index("s")
    wid = cid * NS + sid
    nchunk = src_hbm.shape[1]
    pltpu.sync_copy(src_hbm.at[wid], srcb)
    pltpu.sync_copy(dst_hbm.at[wid], dstb)
    pltpu.sync_copy(w_hbm.at[wid], wb)
    pltpu.sync_copy(znd_hbm.at[pl.ds(sid * SL, SL)],
                    accs.at[pl.ds(sid * SL, SL)])
    plsc.subcore_barrier()

    @pl.loop(0, nchunk)
    def _(c):
        pltpu.sync_copy(g_hbm.at[srcb.at[c]], rows)

        @pl.loop(0, K)
        def _(i):
            wv = plsc.load_gather(
                wb, [jnp.full((16,), c, jnp.int32), jnp.full((16,), i, jnp.int32)])
            for j in range(D // 16):
                sl = (i, pl.ds(j * 16, 16))
                rows[sl] = rows[sl] * wv

        pltpu.sync_copy(rows, accs.at[dstb.at[c]], add=True)

    plsc.subcore_barrier()
    pltpu.sync_copy(accs.at[pl.ds(sid * SL, SL)],
                    out_hbm.at[cid, pl.ds(sid * SL, SL)])


def _sc_aggregate(src3, dst3, w3, g, nchunk):
    kern = functools.partial(
        pl.kernel,
        out_type=jax.ShapeDtypeStruct((NC, NP, D), jnp.float32),
        mesh=_mesh,
        scratch_types=[
            pltpu.VMEM((nchunk, K), jnp.int32),
            pltpu.VMEM((nchunk, K), jnp.int32),
            pltpu.VMEM((nchunk, K), jnp.float32),
            pltpu.VMEM((K, D), jnp.float32),
            pltpu.VMEM_SHARED((NP, D), jnp.float32),
        ],
        compiler_params=_sc_params,
    )(_agg_body)
    znd = jnp.zeros((NP, D), jnp.float32)
    return kern(src3, dst3, w3, g, znd)


_BN = 400


def _mm_body(x_ref, w_ref, o_ref):
    o_ref[...] = jnp.dot(x_ref[...], w_ref[...],
                         preferred_element_type=jnp.float32,
                         precision=lax.Precision.HIGHEST)


def _tc_matmul(x, W):
    return pl.pallas_call(
        _mm_body,
        grid=(N // _BN,),
        in_specs=[
            pl.BlockSpec((_BN, D), lambda i: (i, 0)),
            pl.BlockSpec((D, D), lambda i: (0, 0)),
        ],
        out_specs=pl.BlockSpec((_BN, D), lambda i: (i, 0)),
        out_shape=jax.ShapeDtypeStruct((N, D), jnp.float32),
    )(x, W)


def _scale_body(degp_ref, h_ref, o_ref):
    deg = degp_ref[:, 0] + degp_ref[:, 1] + 1.0
    dinv = jnp.where(deg > 0, lax.rsqrt(deg), 0.0)
    o_ref[...] = h_ref[...] * dinv[:, None]


def _tc_scale(degp, h):
    return pl.pallas_call(
        _scale_body,
        grid=(N // _BN,),
        in_specs=[
            pl.BlockSpec((_BN, NC), lambda i: (i, 0)),
            pl.BlockSpec((_BN, D), lambda i: (i, 0)),
        ],
        out_specs=pl.BlockSpec((_BN, D), lambda i: (i, 0)),
        out_shape=jax.ShapeDtypeStruct((N, D), jnp.float32),
    )(degp, h)


def _final_body(degp_ref, accp_ref, g_ref, b_ref, o_ref):
    deg = degp_ref[:, 0] + degp_ref[:, 1] + 1.0
    dinv = jnp.where(deg > 0, lax.rsqrt(deg), 0.0)
    acc = accp_ref[0] + accp_ref[1] + g_ref[...]
    o_ref[...] = acc * dinv[:, None] + b_ref[...]


def _tc_final(degp, accp, g, b):
    return pl.pallas_call(
        _final_body,
        grid=(N // _BN,),
        in_specs=[
            pl.BlockSpec((_BN, NC), lambda i: (i, 0)),
            pl.BlockSpec((NC, _BN, D), lambda i: (0, i, 0)),
            pl.BlockSpec((_BN, D), lambda i: (i, 0)),
            pl.BlockSpec((1, D), lambda i: (0, 0)),
        ],
        out_specs=pl.BlockSpec((_BN, D), lambda i: (i, 0)),
        out_shape=jax.ShapeDtypeStruct((N, D), jnp.float32),
    )(degp, accp, g, b)


def kernel(x, edge_index, edge_attr, W, b):
    E = edge_index.shape[1]
    per_tile = -(-E // (NW * K)) * K
    EP = per_tile * NW
    nchunk = per_tile // K
    pad = EP - E

    src = jnp.concatenate([edge_index[0], jnp.zeros((pad,), jnp.int32)])
    dst = jnp.concatenate([edge_index[1], jnp.zeros((pad,), jnp.int32)])
    w = jnp.concatenate([edge_attr, jnp.zeros((pad,), jnp.float32)])
    src3 = src.reshape(NW, nchunk, K)
    dst3 = dst.reshape(NW, nchunk, K)
    w3 = w.reshape(NW, nchunk, K)

    degp = _sc_degree(dst3, w3, nchunk)
    h = _tc_matmul(x, W)
    degp_t = degp[:, :N].T
    g = _tc_scale(degp_t, h)
    accp = _sc_aggregate(src3, dst3, w3, g, nchunk)
    out = _tc_final(degp_t, accp[:, :N, :], g, b.reshape(1, D))

    out = out.reshape(N, SEQ, D // SEQ)
    out = jnp.transpose(out, (1, 0, 2))
    return out[None]
